# baseline (speedup 1.0000x reference)
import functools

import jax
import jax.numpy as jnp
from jax import lax
from jax.experimental import pallas as pl
from jax.experimental.pallas import tpu as pltpu
from jax.experimental.pallas import tpu_sc as plsc

B, S = 1024, 50
VOCAB = 1000
ROWS = B * S
NC, NS, L = 2, 16, 16
NW = NC * NS
RPW = ROWS // NW
CH = 32
NCHUNK = RPW // CH

_mesh = plsc.VectorSubcoreMesh(core_axis_name="c", subcore_axis_name="s")


@functools.partial(
    pl.kernel,
    mesh=_mesh,
    out_type=jax.ShapeDtypeStruct((ROWS * VOCAB,), jnp.float32),
    scratch_types=[
        pltpu.VMEM((RPW,), jnp.int32),
        pltpu.VMEM((CH * VOCAB,), jnp.float32),
        pltpu.VMEM((CH * VOCAB,), jnp.float32),
        pltpu.SemaphoreType.DMA,
        pltpu.SemaphoreType.DMA,
    ],
    compiler_params=pltpu.CompilerParams(needs_layout_passes=False),
)
def _onehot_sc(x_hbm, out_hbm, idx_v, buf0, buf1, sem0, sem1):
    wid = lax.axis_index("s") * NC + lax.axis_index("c")
    base_row = wid * RPW
    pltpu.sync_copy(x_hbm.at[pl.ds(base_row, RPW)], idx_v)

    zeros = jnp.zeros((L,), jnp.float32)
    ones = jnp.ones((L,), jnp.float32)
    bufs = ((buf0, sem0), (buf1, sem1))

    def zero_body(i, carry):
        buf0[pl.ds(i * L, L)] = zeros
        buf1[pl.ds(i * L, L)] = zeros
        return carry

    lax.fori_loop(0, CH * VOCAB // L, zero_body, 0)

    def offsets(c, g):
        idx_vec = idx_v[pl.ds(c * CH + g * L, L)]
        local_row = lax.iota(jnp.int32, L) + g * L
        return local_row * VOCAB + idx_vec

    def chunk_body(c2, carry):
        for b, (buf, sem) in enumerate(bufs):
            c = 2 * c2 + b

            @pl.when(c2 > 0)
            def _():
                pltpu.make_async_copy(
                    buf, out_hbm.at[pl.ds(0, CH * VOCAB)], sem
                ).wait()
                for g in range(CH // L):
                    plsc.store_scatter(buf, [offsets(c - 2, g)], zeros)

            for g in range(CH // L):
                plsc.store_scatter(buf, [offsets(c, g)], ones)
            dst = out_hbm.at[pl.ds((base_row + c * CH) * VOCAB, CH * VOCAB)]
            pltpu.async_copy(buf, dst, sem)
        return carry

    lax.fori_loop(0, NCHUNK // 2, chunk_body, 0)
    for buf, sem in bufs:
        pltpu.make_async_copy(buf, out_hbm.at[pl.ds(0, CH * VOCAB)], sem).wait()


def kernel(X):
    xflat = X.reshape(-1).astype(jnp.int32)
    out = _onehot_sc(xflat)
    return out.reshape(B, S, VOCAB)


# speedup vs baseline: 1.9024x; 1.9024x over previous
import functools

import jax
import jax.numpy as jnp
from jax import lax
from jax.experimental import pallas as pl
from jax.experimental.pallas import tpu as pltpu
from jax.experimental.pallas import tpu_sc as plsc

B, S = 1024, 50
VOCAB = 1000
NC, NS, L = 2, 16, 16
NW = NC * NS
BPW = B // NW
RPW = BPW * S
NG = (S + L - 1) // L

_mesh = plsc.VectorSubcoreMesh(core_axis_name="c", subcore_axis_name="s")


@functools.partial(
    pl.kernel,
    mesh=_mesh,
    out_type=jax.ShapeDtypeStruct((B, S, VOCAB), jnp.float32),
    scratch_types=[
        pltpu.VMEM((RPW,), jnp.int32),
        pltpu.VMEM((S, VOCAB), jnp.float32),
        pltpu.VMEM((S, VOCAB), jnp.float32),
        pltpu.SemaphoreType.DMA,
        pltpu.SemaphoreType.DMA,
    ],
    compiler_params=pltpu.CompilerParams(needs_layout_passes=False),
)
def _onehot_sc(x_hbm, out_hbm, idx_v, buf0, buf1, sem0, sem1):
    wid = lax.axis_index("s") * NC + lax.axis_index("c")
    b0 = wid * BPW
    pltpu.sync_copy(x_hbm.at[pl.ds(b0 * S, RPW)], idx_v)

    zeros = jnp.zeros((L,), jnp.float32)
    ones = jnp.ones((L,), jnp.float32)
    lane = lax.iota(jnp.int32, L)
    bufs = ((buf0, sem0), (buf1, sem1))

    def zero_s(s, carry):
        for k in range(VOCAB // L):
            buf0[s, pl.ds(k * L, L)] = zeros
            buf1[s, pl.ds(k * L, L)] = zeros
        buf0[s, pl.ds(VOCAB - L, L)] = zeros
        buf1[s, pl.ds(VOCAB - L, L)] = zeros
        return carry

    lax.fori_loop(0, S, zero_s, 0)

    def put(buf, bl, val):
        for g in range(NG):
            pos = jnp.minimum(bl * S + g * L + lane, RPW - 1)
            v_vec = plsc.load_gather(idx_v, [pos])
            s_vec = lane + g * L
            if (g + 1) * L > S:
                plsc.store_scatter(buf, [s_vec, v_vec], val, mask=lane < S - g * L)
            else:
                plsc.store_scatter(buf, [s_vec, v_vec], val)

    def chunk_body(c2, carry):
        for i, (buf, sem) in enumerate(bufs):
            bl = 2 * c2 + i

            @pl.when(c2 > 0)
            def _():
                pltpu.make_async_copy(buf, out_hbm.at[0], sem).wait()
                put(buf, bl - 2, zeros)

            put(buf, bl, ones)
            pltpu.async_copy(buf, out_hbm.at[b0 + bl], sem)
        return carry

    lax.fori_loop(0, BPW // 2, chunk_body, 0)
    for buf, sem in bufs:
        pltpu.make_async_copy(buf, out_hbm.at[0], sem).wait()


def kernel(X):
    xflat = X.reshape(-1).astype(jnp.int32)
    return _onehot_sc(xflat)


# speedup vs baseline: 6.8437x; 3.5973x over previous
import functools

import jax
import jax.numpy as jnp
from jax import lax
from jax.experimental import pallas as pl
from jax.experimental.pallas import tpu as pltpu
from jax.experimental.pallas import tpu_sc as plsc

B, S = 1024, 50
VOCAB = 1000
NC, NS, L = 2, 16, 16
NW = NC * NS
NCOL = B // 128
NSSUB = NW // NCOL
CV = 200
NVC = VOCAB // CV
NRING = 2

_mesh = plsc.VectorSubcoreMesh(core_axis_name="c", subcore_axis_name="s")


@functools.partial(
    pl.kernel,
    mesh=_mesh,
    out_type=jax.ShapeDtypeStruct((S, VOCAB, B), jnp.float32),
    scratch_types=[
        pltpu.VMEM((128 * S,), jnp.int32),
        *([pltpu.VMEM((CV, 128), jnp.float32)] * NRING),
        *([pltpu.SemaphoreType.DMA] * NRING),
    ],
    compiler_params=pltpu.CompilerParams(needs_layout_passes=False),
)
def _onehot_sc(x_hbm, out_hbm, idx_v, *bufs_sems):
    bufs = tuple(zip(bufs_sems[:NRING], bufs_sems[NRING:]))
    wid = lax.axis_index("s") * NC + lax.axis_index("c")
    col = wid % NCOL
    r = wid // NCOL
    ns = jnp.where(r < NSSUB // 2, (S + NSSUB - 1) // NSSUB, S // NSSUB)
    nch = ns * NVC

    pltpu.sync_copy(x_hbm.at[pl.ds(col * 128 * S, 128 * S)], idx_v)

    zeros = jnp.zeros((L,), jnp.float32)
    ones = jnp.ones((L,), jnp.float32)
    lane = lax.iota(jnp.int32, L)

    def zero_body(i, carry):
        for buf, _ in bufs:
            for g in range(128 // L):
                buf[i, pl.ds(g * L, L)] = zeros
        return carry

    lax.fori_loop(0, CV, zero_body, 0)

    def put(buf, cid, val):
        s = (cid // NVC) * NSSUB + r
        v0 = (cid % NVC) * CV
        for g in range(128 // L):
            bl = g * L + lane
            v_vec = plsc.load_gather(idx_v, [bl * S + s])
            m = (v_vec >= v0) & (v_vec < v0 + CV)
            plsc.store_scatter(buf, [v_vec - v0, bl], val, mask=m)

    def chunk_body(i, carry):
        for slot, (buf, sem) in enumerate(bufs):
            cid = NRING * i + slot

            @pl.when(cid < nch)
            def _():
                @pl.when(cid >= NRING)
                def _():
                    pltpu.make_async_copy(buf, out_hbm.at[0].at[pl.ds(0, CV), pl.ds(0, 128)], sem).wait()
                    put(buf, cid - NRING, zeros)

                put(buf, cid, ones)
                s = (cid // NVC) * NSSUB + r
                v0 = (cid % NVC) * CV
                dst = out_hbm.at[s].at[pl.ds(v0, CV), pl.ds(col * 128, 128)]
                pltpu.async_copy(buf, dst, sem)
        return carry

    max_nch = ((S + NSSUB - 1) // NSSUB) * NVC
    lax.fori_loop(0, (max_nch + NRING - 1) // NRING, chunk_body, 0)
    for buf, sem in bufs:
        pltpu.make_async_copy(buf, out_hbm.at[0].at[pl.ds(0, CV), pl.ds(0, 128)], sem).wait()


def kernel(X):
    xflat = X.reshape(-1).astype(jnp.int32)
    return jnp.transpose(_onehot_sc(xflat), (2, 0, 1))
